# baseline (speedup 1.0000x reference)
import functools

import jax
import jax.numpy as jnp
from jax import lax
from jax.experimental import pallas as pl
from jax.experimental.pallas import tpu as pltpu
from jax.experimental.pallas import tpu_sc as plsc

N_NODES = 10000
NODE_DIM = 128
N_EDGES = 320000
HID = 256

NC = 2
NS = 16
NW = NC * NS

ROWS = N_EDGES // 128
ROWS_PAD = 2560
E_PAD = ROWS_PAD * 128

_mesh = plsc.VectorSubcoreMesh(core_axis_name="c", subcore_axis_name="s")


_C = 64
_STEPS = 160
_IDXROWS = E_PAD // _C
_IB = 16
_NBLK = _STEPS // _IB


@functools.partial(
    pl.kernel,
    out_type=(
        jax.ShapeDtypeStruct((E_PAD, NODE_DIM), jnp.float32),
        jax.ShapeDtypeStruct((E_PAD, NODE_DIM), jnp.float32),
    ),
    mesh=_mesh,
    scratch_types=[
        pltpu.VMEM((2, _IB, _C), jnp.int32),
        pltpu.VMEM((2, _IB, _C), jnp.int32),
        pltpu.VMEM((2, _C, NODE_DIM), jnp.float32),
        pltpu.VMEM((2, _C, NODE_DIM), jnp.float32),
        pltpu.VMEM_SHARED((N_NODES, NODE_DIM), jnp.float32),
    ] + [pltpu.SemaphoreType.DMA] * 6,
)
def _gather_sc(x_hbm, idxi_hbm, idxj_hbm, hi_hbm, hj_hbm,
               idxi_v, idxj_v, bi, bj, x_sh, *sems):
    sem_ix = sems[0:2]
    sem_g = sems[2:4]
    sem_w = sems[4:6]
    cid = lax.axis_index("c")
    sid = lax.axis_index("s")
    wid = sid * NC + cid
    row0 = wid * _STEPS

    @pl.when(sid < 15)
    def _():
        pltpu.sync_copy(x_hbm.at[pl.ds(sid * 632, 632)],
                        x_sh.at[pl.ds(sid * 632, 632)])

    @pl.when(sid == 15)
    def _():
        pltpu.sync_copy(x_hbm.at[pl.ds(9480, 520)],
                        x_sh.at[pl.ds(9480, 520)])

    def fire_ix(b, slot):
        pltpu.async_copy(idxi_hbm.at[pl.ds(row0 + b * _IB, _IB)],
                         idxi_v.at[slot], sem_ix[slot])
        pltpu.async_copy(idxj_hbm.at[pl.ds(row0 + b * _IB, _IB)],
                         idxj_v.at[slot], sem_ix[slot])

    def wait_ix(b, slot):
        pltpu.make_async_copy(idxi_hbm.at[pl.ds(row0 + b * _IB, _IB)],
                              idxi_v.at[slot], sem_ix[slot]).wait()
        pltpu.make_async_copy(idxj_hbm.at[pl.ds(row0 + b * _IB, _IB)],
                              idxj_v.at[slot], sem_ix[slot]).wait()

    def fire_g(slot, t, s):
        pltpu.async_copy(x_sh.at[idxi_v.at[slot, t]], bi.at[s], sem_g[s])
        pltpu.async_copy(x_sh.at[idxj_v.at[slot, t]], bj.at[s], sem_g[s])

    def wait_g(slot, t, s):
        pltpu.make_async_copy(x_sh.at[idxi_v.at[slot, t]], bi.at[s], sem_g[s]).wait()
        pltpu.make_async_copy(x_sh.at[idxj_v.at[slot, t]], bj.at[s], sem_g[s]).wait()

    def fire_w(k, s):
        pltpu.async_copy(bi.at[s], hi_hbm.at[pl.ds((row0 + k) * _C, _C)], sem_w[s])
        pltpu.async_copy(bj.at[s], hj_hbm.at[pl.ds((row0 + k) * _C, _C)], sem_w[s])

    def wait_w(k, s):
        pltpu.make_async_copy(bi.at[s], hi_hbm.at[pl.ds((row0 + k) * _C, _C)], sem_w[s]).wait()
        pltpu.make_async_copy(bj.at[s], hj_hbm.at[pl.ds((row0 + k) * _C, _C)], sem_w[s]).wait()

    fire_ix(0, 0)
    fire_ix(1, 1)
    plsc.subcore_barrier()

    def body(bb, carry):
        for sb in range(2):
            b = bb * 2 + sb
            wait_ix(b, sb)

            for t in range(_IB):
                k = b * _IB + t
                s = t % 2

                @pl.when(k >= 2)
                def _():
                    wait_w(k - 2, s)

                fire_g(sb, t, s)
                wait_g(sb, t, s)
                fire_w(k, s)

            @pl.when(b < _NBLK - 2)
            def _():
                fire_ix(b + 2, sb)
        return carry

    lax.fori_loop(0, _NBLK // 2, body, 0)
    wait_w(_STEPS - 2, 0)
    wait_w(_STEPS - 1, 1)


@functools.partial(
    pl.kernel,
    out_type=(
        jax.ShapeDtypeStruct((N_NODES, NODE_DIM), jnp.float32),
        jax.ShapeDtypeStruct((N_NODES, NODE_DIM), jnp.float32),
    ),
    mesh=_mesh,
    scratch_types=[
        pltpu.VMEM((80, 128), jnp.int32),
        pltpu.VMEM((128, NODE_DIM), jnp.float32),
        pltpu.VMEM_SHARED((N_NODES, NODE_DIM), jnp.float32),
        pltpu.SemaphoreType.DMA,
    ],
)
def _scatter_sc(ea_hbm, idxj_hbm, zeros_hbm, p0_hbm, p1_hbm,
                idx_v, rows_v, shared, sem):
    cid = lax.axis_index("c")
    sid = lax.axis_index("s")
    wid = sid * NC + cid

    @pl.when(sid == 0)
    def _():
        pltpu.sync_copy(zeros_hbm, shared)

    plsc.subcore_barrier()

    row0 = wid * 80

    @pl.when(wid < 31)
    def _():
        pltpu.sync_copy(idxj_hbm.at[pl.ds(row0, 80)], idx_v)

    @pl.when(wid == 31)
    def _():
        pltpu.sync_copy(idxj_hbm.at[pl.ds(2480, 20)], idx_v.at[pl.ds(0, 20)])

    cnt = jnp.where(wid < 31, 80, 20)

    def body(k, carry):
        pltpu.sync_copy(ea_hbm.at[pl.ds((row0 + k) * 128, 128)], rows_v)
        pltpu.sync_copy(rows_v, shared.at[idx_v.at[k]], add=True)
        return carry

    lax.fori_loop(0, cnt, body, 0)

    plsc.subcore_barrier()

    @pl.when(jnp.logical_and(cid == 0, sid < 15))
    def _():
        pltpu.sync_copy(shared.at[pl.ds(sid * 632, 632)],
                        p0_hbm.at[pl.ds(sid * 632, 632)])

    @pl.when(jnp.logical_and(cid == 0, sid == 15))
    def _():
        pltpu.sync_copy(shared.at[pl.ds(9480, 520)],
                        p0_hbm.at[pl.ds(9480, 520)])

    @pl.when(jnp.logical_and(cid == 1, sid < 15))
    def _():
        pltpu.sync_copy(shared.at[pl.ds(sid * 632, 632)],
                        p1_hbm.at[pl.ds(sid * 632, 632)])

    @pl.when(jnp.logical_and(cid == 1, sid == 15))
    def _():
        pltpu.sync_copy(shared.at[pl.ds(9480, 520)],
                        p1_hbm.at[pl.ds(9480, 520)])


def _edge_body(hi_ref, hj_ref, ea_ref, w1_ref, b1_ref, w2_ref, b2_ref,
               g_ref, b_ref, out_ref):
    hi = hi_ref[...].astype(jnp.bfloat16)
    hj = hj_ref[...].astype(jnp.bfloat16)
    ea = ea_ref[...]
    w1 = w1_ref[...].astype(jnp.bfloat16)
    h = (jnp.dot(hi, w1[:128], preferred_element_type=jnp.float32)
         + jnp.dot(hj, w1[128:256], preferred_element_type=jnp.float32)
         + jnp.dot(ea.astype(jnp.bfloat16), w1[256:384],
                   preferred_element_type=jnp.float32)
         + b1_ref[...])
    h = jnp.maximum(h, 0.0)
    o = jnp.dot(h.astype(jnp.bfloat16), w2_ref[...].astype(jnp.bfloat16),
                preferred_element_type=jnp.float32) + b2_ref[...]
    mu = jnp.mean(o, axis=-1, keepdims=True)
    var = jnp.mean((o - mu) ** 2, axis=-1, keepdims=True)
    o = (o - mu) * lax.rsqrt(var + 1e-5) * g_ref[...] + b_ref[...]
    out_ref[...] = ea + o


def _edge_mlp(hi, hj, ea, w1, b1, w2, b2, g, b):
    BE = 1024
    grid = (N_EDGES + BE - 1) // BE
    return pl.pallas_call(
        _edge_body,
        grid=(grid,),
        in_specs=[
            pl.BlockSpec((BE, 128), lambda i: (i, 0)),
            pl.BlockSpec((BE, 128), lambda i: (i, 0)),
            pl.BlockSpec((BE, 128), lambda i: (i, 0)),
            pl.BlockSpec((384, 256), lambda i: (0, 0)),
            pl.BlockSpec((1, 256), lambda i: (0, 0)),
            pl.BlockSpec((256, 128), lambda i: (0, 0)),
            pl.BlockSpec((1, 128), lambda i: (0, 0)),
            pl.BlockSpec((1, 128), lambda i: (0, 0)),
            pl.BlockSpec((1, 128), lambda i: (0, 0)),
        ],
        out_specs=pl.BlockSpec((BE, 128), lambda i: (i, 0)),
        out_shape=jax.ShapeDtypeStruct((N_EDGES, 128), jnp.float32),
    )(hi, hj, ea, w1, b1, w2, b2, g, b)


def _node_body(x_ref, p0_ref, p1_ref, w1_ref, b1_ref, w2_ref, b2_ref,
               g_ref, b_ref, out_ref):
    x = x_ref[...]
    agg = p0_ref[...] + p1_ref[...]
    w1 = w1_ref[...]
    h = (jnp.dot(x, w1[:128], preferred_element_type=jnp.float32)
         + jnp.dot(agg, w1[128:256], preferred_element_type=jnp.float32)
         + b1_ref[...])
    h = jnp.maximum(h, 0.0)
    o = jnp.dot(h, w2_ref[...], preferred_element_type=jnp.float32) + b2_ref[...]
    mu = jnp.mean(o, axis=-1, keepdims=True)
    var = jnp.mean((o - mu) ** 2, axis=-1, keepdims=True)
    o = (o - mu) * lax.rsqrt(var + 1e-5) * g_ref[...] + b_ref[...]
    out_ref[...] = x + o


def _node_mlp(x, p0, p1, w1, b1, w2, b2, g, b):
    BN = 2000
    grid = N_NODES // BN
    return pl.pallas_call(
        _node_body,
        grid=(grid,),
        in_specs=[
            pl.BlockSpec((BN, 128), lambda i: (i, 0)),
            pl.BlockSpec((BN, 128), lambda i: (i, 0)),
            pl.BlockSpec((BN, 128), lambda i: (i, 0)),
            pl.BlockSpec((256, 256), lambda i: (0, 0)),
            pl.BlockSpec((1, 256), lambda i: (0, 0)),
            pl.BlockSpec((256, 128), lambda i: (0, 0)),
            pl.BlockSpec((1, 128), lambda i: (0, 0)),
            pl.BlockSpec((1, 128), lambda i: (0, 0)),
            pl.BlockSpec((1, 128), lambda i: (0, 0)),
        ],
        out_specs=pl.BlockSpec((BN, 128), lambda i: (i, 0)),
        out_shape=jax.ShapeDtypeStruct((N_NODES, 128), jnp.float32),
    )(x, p0, p1, w1, b1, w2, b2, g, b)


def kernel(x, edge_index, edge_attr, eW1, eb1, eW2, eb2, e_ln_g, e_ln_b,
           nW1, nb1, nW2, nb2, n_ln_g, n_ln_b):
    ei = edge_index.astype(jnp.int32)
    ei_pad = jnp.pad(ei, ((0, 0), (0, E_PAD - N_EDGES)))
    idxi = ei_pad[0].reshape(_IDXROWS, _C)
    idxj = ei_pad[1].reshape(_IDXROWS, _C)

    hi, hj = _gather_sc(x, idxi, idxj)

    new_ea = _edge_mlp(hi, hj, edge_attr, eW1, eb1.reshape(1, -1),
                       eW2, eb2.reshape(1, -1),
                       e_ln_g.reshape(1, -1), e_ln_b.reshape(1, -1))

    idxj_real = ei[1].reshape(ROWS, 128)
    zeros = jnp.zeros((N_NODES, NODE_DIM), jnp.float32)
    p0, p1 = _scatter_sc(new_ea, idxj_real, zeros)

    new_x = _node_mlp(x, p0, p1, nW1, nb1.reshape(1, -1),
                      nW2, nb2.reshape(1, -1),
                      n_ln_g.reshape(1, -1), n_ln_b.reshape(1, -1))
    return new_x, new_ea


# speedup vs baseline: 1.3093x; 1.3093x over previous
import functools

import jax
import jax.numpy as jnp
from jax import lax
from jax.experimental import pallas as pl
from jax.experimental.pallas import tpu as pltpu
from jax.experimental.pallas import tpu_sc as plsc

N_NODES = 10000
NODE_DIM = 128
PACK_DIM = NODE_DIM // 2
N_EDGES = 320000
HID = 256

NC = 2
NS = 16
NW = NC * NS

ROWS = N_EDGES // 128
ROWS_PAD = 2560
E_PAD = ROWS_PAD * 128

_mesh = plsc.VectorSubcoreMesh(core_axis_name="c", subcore_axis_name="s")


_C = 64
_STEPS = 160
_IB = 16
_NBLK = _STEPS // _IB


@functools.partial(
    pl.kernel,
    out_type=(
        jax.ShapeDtypeStruct((E_PAD, NODE_DIM), jnp.float32),
        jax.ShapeDtypeStruct((E_PAD, NODE_DIM), jnp.float32),
    ),
    mesh=_mesh,
    scratch_types=[
        pltpu.VMEM((2, _IB, _C), jnp.int32),
        pltpu.VMEM((2, _IB, _C), jnp.int32),
        pltpu.VMEM((2, _C, NODE_DIM), jnp.float32),
        pltpu.VMEM((2, _C, NODE_DIM), jnp.float32),
        pltpu.VMEM_SHARED((N_NODES, NODE_DIM), jnp.float32),
    ] + [pltpu.SemaphoreType.DMA] * 6,
)
def _gather_sc(x_hbm, idxi_hbm, idxj_hbm, hi_hbm, hj_hbm,
               idxi_v, idxj_v, bi, bj, x_sh, *sems):
    sem_ix = sems[0:2]
    sem_g = sems[2:4]
    sem_w = sems[4:6]
    cid = lax.axis_index("c")
    sid = lax.axis_index("s")
    wid = sid * NC + cid
    row0 = wid * _STEPS

    @pl.when(sid < 15)
    def _():
        pltpu.sync_copy(x_hbm.at[pl.ds(sid * 632, 632)],
                        x_sh.at[pl.ds(sid * 632, 632)])

    @pl.when(sid == 15)
    def _():
        pltpu.sync_copy(x_hbm.at[pl.ds(9480, 520)],
                        x_sh.at[pl.ds(9480, 520)])

    def fire_ix(b, slot):
        pltpu.async_copy(idxi_hbm.at[pl.ds(row0 + b * _IB, _IB)],
                         idxi_v.at[slot], sem_ix[slot])
        pltpu.async_copy(idxj_hbm.at[pl.ds(row0 + b * _IB, _IB)],
                         idxj_v.at[slot], sem_ix[slot])

    def wait_ix(b, slot):
        pltpu.make_async_copy(idxi_hbm.at[pl.ds(row0 + b * _IB, _IB)],
                              idxi_v.at[slot], sem_ix[slot]).wait()
        pltpu.make_async_copy(idxj_hbm.at[pl.ds(row0 + b * _IB, _IB)],
                              idxj_v.at[slot], sem_ix[slot]).wait()

    def fire_g(slot, t, s):
        pltpu.async_copy(x_sh.at[idxi_v.at[slot, t]], bi.at[s], sem_g[s])
        pltpu.async_copy(x_sh.at[idxj_v.at[slot, t]], bj.at[s], sem_g[s])

    def wait_g(slot, t, s):
        pltpu.make_async_copy(x_sh.at[idxi_v.at[slot, t]], bi.at[s], sem_g[s]).wait()
        pltpu.make_async_copy(x_sh.at[idxj_v.at[slot, t]], bj.at[s], sem_g[s]).wait()

    def fire_w(k, s):
        pltpu.async_copy(bi.at[s], hi_hbm.at[pl.ds((row0 + k) * _C, _C)], sem_w[s])
        pltpu.async_copy(bj.at[s], hj_hbm.at[pl.ds((row0 + k) * _C, _C)], sem_w[s])

    def wait_w(k, s):
        pltpu.make_async_copy(bi.at[s], hi_hbm.at[pl.ds((row0 + k) * _C, _C)], sem_w[s]).wait()
        pltpu.make_async_copy(bj.at[s], hj_hbm.at[pl.ds((row0 + k) * _C, _C)], sem_w[s]).wait()

    fire_ix(0, 0)
    fire_ix(1, 1)
    plsc.subcore_barrier()

    def body(bb, carry):
        for sb in range(2):
            b = bb * 2 + sb
            wait_ix(b, sb)

            for t in range(_IB):
                k = b * _IB + t
                s = t % 2

                @pl.when(k >= 2)
                def _():
                    wait_w(k - 2, s)

                fire_g(sb, t, s)
                wait_g(sb, t, s)
                fire_w(k, s)

            @pl.when(b < _NBLK - 2)
            def _():
                fire_ix(b + 2, sb)
        return carry

    lax.fori_loop(0, _NBLK // 2, body, 0)
    wait_w(_STEPS - 2, 0)
    wait_w(_STEPS - 1, 1)


@functools.partial(
    pl.kernel,
    out_type=(
        jax.ShapeDtypeStruct((N_NODES, NODE_DIM), jnp.float32),
        jax.ShapeDtypeStruct((N_NODES, NODE_DIM), jnp.float32),
    ),
    mesh=_mesh,
    scratch_types=[
        pltpu.VMEM((80, 128), jnp.int32),
        pltpu.VMEM((2, 128, NODE_DIM), jnp.float32),
        pltpu.VMEM_SHARED((N_NODES, NODE_DIM), jnp.float32),
    ] + [pltpu.SemaphoreType.DMA] * 4,
)
def _scatter_sc(ea_hbm, idxj_hbm, zeros_hbm, p0_hbm, p1_hbm,
                idx_v, rows_v, shared, *sems):
    sem_l = sems[0:2]
    sem_a = sems[2:4]
    cid = lax.axis_index("c")
    sid = lax.axis_index("s")
    wid = sid * NC + cid

    @pl.when(sid == 0)
    def _():
        pltpu.sync_copy(zeros_hbm, shared)

    plsc.subcore_barrier()

    row0 = wid * 80

    @pl.when(wid < 31)
    def _():
        pltpu.sync_copy(idxj_hbm.at[pl.ds(row0, 80)], idx_v)

    @pl.when(wid == 31)
    def _():
        pltpu.sync_copy(idxj_hbm.at[pl.ds(2480, 20)], idx_v.at[pl.ds(0, 20)])

    cnt = jnp.where(wid < 31, 80, 20)

    def fire_l(k, s):
        pltpu.async_copy(ea_hbm.at[pl.ds((row0 + k) * 128, 128)],
                         rows_v.at[s], sem_l[s])

    def wait_l(k, s):
        pltpu.make_async_copy(ea_hbm.at[pl.ds((row0 + k) * 128, 128)],
                              rows_v.at[s], sem_l[s]).wait()

    fire_l(0, 0)

    def body(r, carry):
        for s in range(2):
            k = r * 2 + s

            @pl.when(k + 1 < cnt)
            def _():
                fire_l(k + 1, 1 - s)

            wait_l(k, s)
            pltpu.sync_copy(rows_v.at[s], shared.at[idx_v.at[k]], add=True)
        return carry

    lax.fori_loop(0, cnt // 2, body, 0)

    plsc.subcore_barrier()

    @pl.when(jnp.logical_and(cid == 0, sid < 15))
    def _():
        pltpu.sync_copy(shared.at[pl.ds(sid * 632, 632)],
                        p0_hbm.at[pl.ds(sid * 632, 632)])

    @pl.when(jnp.logical_and(cid == 0, sid == 15))
    def _():
        pltpu.sync_copy(shared.at[pl.ds(9480, 520)],
                        p0_hbm.at[pl.ds(9480, 520)])

    @pl.when(jnp.logical_and(cid == 1, sid < 15))
    def _():
        pltpu.sync_copy(shared.at[pl.ds(sid * 632, 632)],
                        p1_hbm.at[pl.ds(sid * 632, 632)])

    @pl.when(jnp.logical_and(cid == 1, sid == 15))
    def _():
        pltpu.sync_copy(shared.at[pl.ds(9480, 520)],
                        p1_hbm.at[pl.ds(9480, 520)])


def _edge_body(hi_ref, hj_ref, ea_ref, w1_ref, b1_ref, w2_ref, b2_ref,
               g_ref, b_ref, out_ref):
    hi = hi_ref[...].astype(jnp.bfloat16)
    hj = hj_ref[...].astype(jnp.bfloat16)
    ea = ea_ref[...]
    w1 = w1_ref[...].astype(jnp.bfloat16)
    h = (jnp.dot(hi, w1[0:128], preferred_element_type=jnp.float32)
         + jnp.dot(hj, w1[128:256], preferred_element_type=jnp.float32)
         + jnp.dot(ea.astype(jnp.bfloat16), w1[256:384],
                   preferred_element_type=jnp.float32)
         + b1_ref[...])
    h = jnp.maximum(h, 0.0)
    o = jnp.dot(h.astype(jnp.bfloat16), w2_ref[...].astype(jnp.bfloat16),
                preferred_element_type=jnp.float32) + b2_ref[...]
    mu = jnp.mean(o, axis=-1, keepdims=True)
    var = jnp.mean((o - mu) ** 2, axis=-1, keepdims=True)
    o = (o - mu) * lax.rsqrt(var + 1e-5) * g_ref[...] + b_ref[...]
    out_ref[...] = ea + o


def _edge_mlp(hi, hj, ea, w1, b1, w2, b2, g, b):
    BE = 2048
    grid = (N_EDGES + BE - 1) // BE
    return pl.pallas_call(
        _edge_body,
        grid=(grid,),
        in_specs=[
            pl.BlockSpec((BE, 128), lambda i: (i, 0)),
            pl.BlockSpec((BE, 128), lambda i: (i, 0)),
            pl.BlockSpec((BE, 128), lambda i: (i, 0)),
            pl.BlockSpec((384, 256), lambda i: (0, 0)),
            pl.BlockSpec((1, 256), lambda i: (0, 0)),
            pl.BlockSpec((256, 128), lambda i: (0, 0)),
            pl.BlockSpec((1, 128), lambda i: (0, 0)),
            pl.BlockSpec((1, 128), lambda i: (0, 0)),
            pl.BlockSpec((1, 128), lambda i: (0, 0)),
        ],
        out_specs=pl.BlockSpec((BE, 128), lambda i: (i, 0)),
        out_shape=jax.ShapeDtypeStruct((N_EDGES, 128), jnp.float32),
    )(hi, hj, ea, w1, b1, w2, b2, g, b)


def _node_body(x_ref, p0_ref, p1_ref, w1_ref, b1_ref, w2_ref, b2_ref,
               g_ref, b_ref, out_ref):
    x = x_ref[...]
    agg = p0_ref[...] + p1_ref[...]
    w1 = w1_ref[...]
    h = (jnp.dot(x, w1[:128], preferred_element_type=jnp.float32)
         + jnp.dot(agg, w1[128:256], preferred_element_type=jnp.float32)
         + b1_ref[...])
    h = jnp.maximum(h, 0.0)
    o = jnp.dot(h, w2_ref[...], preferred_element_type=jnp.float32) + b2_ref[...]
    mu = jnp.mean(o, axis=-1, keepdims=True)
    var = jnp.mean((o - mu) ** 2, axis=-1, keepdims=True)
    o = (o - mu) * lax.rsqrt(var + 1e-5) * g_ref[...] + b_ref[...]
    out_ref[...] = x + o


def _node_mlp(x, p0, p1, w1, b1, w2, b2, g, b):
    BN = 2000
    grid = N_NODES // BN
    return pl.pallas_call(
        _node_body,
        grid=(grid,),
        in_specs=[
            pl.BlockSpec((BN, 128), lambda i: (i, 0)),
            pl.BlockSpec((BN, 128), lambda i: (i, 0)),
            pl.BlockSpec((BN, 128), lambda i: (i, 0)),
            pl.BlockSpec((256, 256), lambda i: (0, 0)),
            pl.BlockSpec((1, 256), lambda i: (0, 0)),
            pl.BlockSpec((256, 128), lambda i: (0, 0)),
            pl.BlockSpec((1, 128), lambda i: (0, 0)),
            pl.BlockSpec((1, 128), lambda i: (0, 0)),
            pl.BlockSpec((1, 128), lambda i: (0, 0)),
        ],
        out_specs=pl.BlockSpec((BN, 128), lambda i: (i, 0)),
        out_shape=jax.ShapeDtypeStruct((N_NODES, 128), jnp.float32),
    )(x, p0, p1, w1, b1, w2, b2, g, b)


def kernel(x, edge_index, edge_attr, eW1, eb1, eW2, eb2, e_ln_g, e_ln_b,
           nW1, nb1, nW2, nb2, n_ln_g, n_ln_b):
    ei = edge_index.astype(jnp.int32)
    ei_pad = jnp.pad(ei, ((0, 0), (0, E_PAD - N_EDGES)))
    idxi = ei_pad[0].reshape(E_PAD // _C, _C)
    idxj = ei_pad[1].reshape(E_PAD // _C, _C)

    hi, hj = _gather_sc(x, idxi, idxj)

    new_ea = _edge_mlp(hi, hj, edge_attr, eW1, eb1.reshape(1, -1),
                       eW2, eb2.reshape(1, -1),
                       e_ln_g.reshape(1, -1), e_ln_b.reshape(1, -1))

    idxj_real = ei[1].reshape(ROWS, 128)
    zeros = jnp.zeros((N_NODES, NODE_DIM), jnp.float32)
    p0, p1 = _scatter_sc(new_ea, idxj_real, zeros)

    new_x = _node_mlp(x, p0, p1, nW1, nb1.reshape(1, -1),
                      nW2, nb2.reshape(1, -1),
                      n_ln_g.reshape(1, -1), n_ln_b.reshape(1, -1))
    return new_x, new_ea
